# baseline (speedup 1.0000x reference)
import functools

import jax
import jax.numpy as jnp
from jax import lax
from jax.experimental import pallas as pl
from jax.experimental.pallas import tpu as pltpu
from jax.experimental.pallas import tpu_sc as plsc

_EVENTS = 256
_FRAMES = 16384
_HIDDEN = 512
_NC = 2
_NS = 16
_NW = _NC * _NS
_NQ = 4
_QTR = _HIDDEN // _NQ
_NFC = _NW // _NQ
_FSC = 4096
_APW = _FSC // _NFC
_BLKF = 64
_NBLKA = _APW // _BLKF
_FPWB = _FRAMES // _NW
_BLKT = 4096
_GT = (_FRAMES - _FSC) // _BLKT
_OFFB = _FSC // _BLKT
_I32MAX = 2147483647
_I32MIN = -2147483648


def _sc_body(feats_hbm, seg_hbm, part_out, sgrid_out, egrid_out,
             fbuf0, fbuf1, segacc_v, segflat_v, starts_v, ends_v, accum_v,
             sem0, sem1):
    c = lax.axis_index("c")
    s = lax.axis_index("s")
    wid = s * _NC + c
    base = (wid // _NQ) * _APW
    col = (wid % _NQ) * _QTR

    zero = jnp.zeros((16,), jnp.float32)

    def _zero_row(i, carry):
        for j in range(_QTR // 16):
            accum_v[i, pl.ds(j * 16, 16)] = zero
        return carry

    lax.fori_loop(0, _EVENTS, _zero_row, 0)

    pltpu.sync_copy(seg_hbm.at[pl.ds(base, _APW)], segacc_v)

    _NCH = _QTR // 16

    def _src(jb):
        return feats_hbm.at[pl.ds(base + jb * _BLKF, _BLKF), pl.ds(col, _QTR)]

    def _accum_block(buf, jb):
        def _group(g, carry):
            rowvec = segacc_v[pl.ds(jb * _BLKF + g * 16, 16)]
            first = rowvec[0]
            last = rowvec[15]

            def _fast():
                @plsc.parallel_loop(0, _NCH, unroll=2)
                def _chunk(j):
                    xs = [buf[g * 16 + k, pl.ds(j * 16, 16)]
                          for k in range(16)]
                    while len(xs) > 1:
                        xs = [xs[i] + xs[i + 1]
                              for i in range(0, len(xs) - 1, 2)] + \
                             (xs[-1:] if len(xs) % 2 else [])
                    plsc.addupdate(accum_v.at[first, pl.ds(j * 16, 16)],
                                   xs[0])

            def _slow():
                for k in range(16):
                    row = rowvec[k]
                    il = g * 16 + k
                    xs = [buf[il, pl.ds(j * 16, 16)] for j in range(_NCH)]
                    for j in range(_NCH):
                        plsc.addupdate(accum_v.at[row, pl.ds(j * 16, 16)],
                                       xs[j])

            lax.cond(first == last, _fast, _slow)
            return carry

        lax.fori_loop(0, _BLKF // 16, _group, 0)

    pltpu.async_copy(_src(0), fbuf0, sem0)

    def _dstep(t, carry):
        jb0 = t * 2
        pltpu.async_copy(_src(jb0 + 1), fbuf1, sem1)
        pltpu.make_async_copy(_src(jb0), fbuf0, sem0).wait()
        _accum_block(fbuf0, jb0)

        @pl.when(t < _NBLKA // 2 - 1)
        def _prefetch():
            pltpu.async_copy(_src(jb0 + 2), fbuf0, sem0)

        pltpu.make_async_copy(_src(jb0 + 1), fbuf1, sem1).wait()
        _accum_block(fbuf1, jb0 + 1)
        return carry

    lax.fori_loop(0, _NBLKA // 2, _dstep, 0)

    pltpu.sync_copy(accum_v, part_out.at[wid])

    bbase = wid * _FPWB
    segflat_v[pl.ds(0, 16)] = jnp.full((16,), -1, jnp.int32)
    pltpu.sync_copy(seg_hbm.at[pl.ds(bbase, _FPWB)],
                    segflat_v.at[pl.ds(16, _FPWB)])
    lane = lax.iota(jnp.int32, 16)
    big = jnp.full((16,), _I32MAX, jnp.int32)
    small = jnp.full((16,), _I32MIN, jnp.int32)
    for r in range(_EVENTS // 16):
        starts_v[pl.ds(r * 16, 16)] = big
        ends_v[pl.ds(r * 16, 16)] = small
    for p in range(0, _FPWB, 16):
        v = segflat_v[pl.ds(16 + p, 16)]
        pv = segflat_v[pl.ds(15 + p, 16)]
        m = v != pv
        fvec = lane + (bbase + p)
        plsc.store_scatter(starts_v, [v], fvec, mask=m)
        plsc.store_scatter(ends_v, [pv], fvec - 1, mask=m & (pv >= 0))
    last_v = segflat_v[pl.ds(16 + _FPWB - 16, 16)]
    plsc.store_scatter(ends_v, [last_v], lane + (bbase + _FPWB - 16),
                       mask=lane == 15)
    pltpu.sync_copy(starts_v, sgrid_out.at[wid])
    pltpu.sync_copy(ends_v, egrid_out.at[wid])


_sc_segsum = functools.partial(
    pl.kernel,
    out_type=(
        jax.ShapeDtypeStruct((_NW, _EVENTS, _QTR), jnp.float32),
        jax.ShapeDtypeStruct((_NW, _EVENTS), jnp.int32),
        jax.ShapeDtypeStruct((_NW, _EVENTS), jnp.int32),
    ),
    mesh=plsc.VectorSubcoreMesh(core_axis_name="c", subcore_axis_name="s",
                                num_cores=_NC, num_subcores=_NS),
    scratch_types=(
        pltpu.VMEM((_BLKF, _QTR), jnp.float32),
        pltpu.VMEM((_BLKF, _QTR), jnp.float32),
        pltpu.VMEM((_APW,), jnp.int32),
        pltpu.VMEM((_FPWB + 32,), jnp.int32),
        pltpu.VMEM((_EVENTS,), jnp.int32),
        pltpu.VMEM((_EVENTS,), jnp.int32),
        pltpu.VMEM((_EVENTS, _QTR), jnp.float32),
        pltpu.SemaphoreType.DMA,
        pltpu.SemaphoreType.DMA,
    ),
    compiler_params=pltpu.CompilerParams(needs_layout_passes=False),
)(_sc_body)


def _mm_body(seg_ref, feats_ref, out_ref):
    i = pl.program_id(0)
    seg_blk = seg_ref[0, :]
    onehot = (lax.broadcasted_iota(jnp.int32, (_EVENTS, _BLKT), 0)
              == seg_blk[None, :]).astype(jnp.float32)
    part = lax.dot_general(onehot, feats_ref[...], (((1,), (0,)), ((), ())),
                           preferred_element_type=jnp.float32)

    @pl.when(i == 0)
    def _init():
        out_ref[...] = part

    @pl.when(i > 0)
    def _acc():
        out_ref[...] += part


_tc_segmm = pl.pallas_call(
    _mm_body,
    grid=(_GT,),
    in_specs=[
        pl.BlockSpec((1, _BLKT), lambda i: (0, _OFFB + i)),
        pl.BlockSpec((_BLKT, _HIDDEN), lambda i: (_OFFB + i, 0)),
    ],
    out_specs=pl.BlockSpec((_EVENTS, _HIDDEN), lambda i: (0, 0)),
    out_shape=jax.ShapeDtypeStruct((_EVENTS, _HIDDEN), jnp.float32),
)


def _tc_body(p_ref, mm_ref, sgrid_ref, egrid_ref, q_ref, sim_ref, ts_ref):
    quarters = [None] * _NQ
    for w in range(_NW):
        qi = w % _NQ
        blk = p_ref[w]
        quarters[qi] = blk if quarters[qi] is None else quarters[qi] + blk
    sums = jnp.concatenate(quarters, axis=1) + mm_ref[...]
    inv = 1.0 / jnp.maximum(
        jnp.sqrt(jnp.sum(sums * sums, axis=1, keepdims=True)), 1e-12)
    en = sums * inv
    q = q_ref[...]
    qinv = 1.0 / jnp.maximum(
        jnp.sqrt(jnp.sum(q * q, axis=1, keepdims=True)), 1e-12)
    qn = q * qinv
    sim_ref[...] = lax.dot_general(qn, en, (((1,), (1,)), ((), ())),
                                   preferred_element_type=jnp.float32)
    st = jnp.min(sgrid_ref[...], axis=0, keepdims=True).astype(jnp.float32)
    et = jnp.max(egrid_ref[...], axis=0, keepdims=True).astype(jnp.float32)
    ts_ref[...] = jnp.concatenate([st, et], axis=0).T / 5.0


_tc_finish = pl.pallas_call(
    _tc_body,
    out_shape=[
        jax.ShapeDtypeStruct((64, _EVENTS), jnp.float32),
        jax.ShapeDtypeStruct((_EVENTS, 2), jnp.float32),
    ],
)


@jax.jit
def kernel(query_feats, video_feats, video_events):
    feats = video_feats[0]
    seg = video_events.astype(jnp.int32)
    sums_tc = _tc_segmm(seg.reshape(1, _FRAMES), feats)
    part, sgrid, egrid = _sc_segsum(feats, seg)
    sim, ts = _tc_finish(part, sums_tc, sgrid, egrid, query_feats)
    return sim, ts


# speedup vs baseline: 1.0719x; 1.0719x over previous
import functools

import jax
import jax.numpy as jnp
from jax import lax
from jax.experimental import pallas as pl
from jax.experimental.pallas import tpu as pltpu
from jax.experimental.pallas import tpu_sc as plsc

_EVENTS = 256
_FRAMES = 16384
_HIDDEN = 512
_NC = 2
_NS = 16
_NW = _NC * _NS
_NQ = 4
_QTR = _HIDDEN // _NQ
_NFC = _NW // _NQ
_FSC = 2048
_APW = _FSC // _NFC
_BLKF = 64
_NBLKA = _APW // _BLKF
_FPWB = _FRAMES // _NW
_BLKT = 2048
_GT = (_FRAMES - _FSC) // _BLKT
_OFFB = _FSC // _BLKT
_I32MAX = 2147483647
_I32MIN = -2147483648


def _sc_body(feats_hbm, seg_hbm, part_out, sgrid_out, egrid_out,
             fbuf0, fbuf1, segacc_v, segflat_v, starts_v, ends_v, accum_v,
             sem0, sem1):
    c = lax.axis_index("c")
    s = lax.axis_index("s")
    wid = s * _NC + c
    base = (wid // _NQ) * _APW
    col = (wid % _NQ) * _QTR

    zero = jnp.zeros((16,), jnp.float32)

    def _zero_row(i, carry):
        for j in range(_QTR // 16):
            accum_v[i, pl.ds(j * 16, 16)] = zero
        return carry

    lax.fori_loop(0, _EVENTS, _zero_row, 0)

    pltpu.sync_copy(seg_hbm.at[pl.ds(base, _APW)], segacc_v)

    _NCH = _QTR // 16

    def _src(jb):
        return feats_hbm.at[pl.ds(base + jb * _BLKF, _BLKF), pl.ds(col, _QTR)]

    def _accum_block(buf, jb):
        def _group(g, carry):
            rowvec = segacc_v[pl.ds(jb * _BLKF + g * 16, 16)]
            first = rowvec[0]
            last = rowvec[15]

            def _fast():
                @plsc.parallel_loop(0, _NCH, unroll=2)
                def _chunk(j):
                    xs = [buf[g * 16 + k, pl.ds(j * 16, 16)]
                          for k in range(16)]
                    while len(xs) > 1:
                        xs = [xs[i] + xs[i + 1]
                              for i in range(0, len(xs) - 1, 2)] + \
                             (xs[-1:] if len(xs) % 2 else [])
                    plsc.addupdate(accum_v.at[first, pl.ds(j * 16, 16)],
                                   xs[0])

            def _slow():
                for k in range(16):
                    row = rowvec[k]
                    il = g * 16 + k
                    xs = [buf[il, pl.ds(j * 16, 16)] for j in range(_NCH)]
                    for j in range(_NCH):
                        plsc.addupdate(accum_v.at[row, pl.ds(j * 16, 16)],
                                       xs[j])

            lax.cond(first == last, _fast, _slow)
            return carry

        lax.fori_loop(0, _BLKF // 16, _group, 0)

    pltpu.async_copy(_src(0), fbuf0, sem0)

    def _dstep(t, carry):
        jb0 = t * 2
        pltpu.async_copy(_src(jb0 + 1), fbuf1, sem1)
        pltpu.make_async_copy(_src(jb0), fbuf0, sem0).wait()
        _accum_block(fbuf0, jb0)

        @pl.when(t < _NBLKA // 2 - 1)
        def _prefetch():
            pltpu.async_copy(_src(jb0 + 2), fbuf0, sem0)

        pltpu.make_async_copy(_src(jb0 + 1), fbuf1, sem1).wait()
        _accum_block(fbuf1, jb0 + 1)
        return carry

    lax.fori_loop(0, _NBLKA // 2, _dstep, 0)

    pltpu.sync_copy(accum_v, part_out.at[wid])

    bbase = wid * _FPWB
    segflat_v[pl.ds(0, 16)] = jnp.full((16,), -1, jnp.int32)
    pltpu.sync_copy(seg_hbm.at[pl.ds(bbase, _FPWB)],
                    segflat_v.at[pl.ds(16, _FPWB)])
    lane = lax.iota(jnp.int32, 16)
    big = jnp.full((16,), _I32MAX, jnp.int32)
    small = jnp.full((16,), _I32MIN, jnp.int32)
    for r in range(_EVENTS // 16):
        starts_v[pl.ds(r * 16, 16)] = big
        ends_v[pl.ds(r * 16, 16)] = small
    for p in range(0, _FPWB, 16):
        v = segflat_v[pl.ds(16 + p, 16)]
        pv = segflat_v[pl.ds(15 + p, 16)]
        m = v != pv
        fvec = lane + (bbase + p)
        plsc.store_scatter(starts_v, [v], fvec, mask=m)
        plsc.store_scatter(ends_v, [pv], fvec - 1, mask=m & (pv >= 0))
    last_v = segflat_v[pl.ds(16 + _FPWB - 16, 16)]
    plsc.store_scatter(ends_v, [last_v], lane + (bbase + _FPWB - 16),
                       mask=lane == 15)
    pltpu.sync_copy(starts_v, sgrid_out.at[wid])
    pltpu.sync_copy(ends_v, egrid_out.at[wid])


_sc_segsum = functools.partial(
    pl.kernel,
    out_type=(
        jax.ShapeDtypeStruct((_NW, _EVENTS, _QTR), jnp.float32),
        jax.ShapeDtypeStruct((_NW, _EVENTS), jnp.int32),
        jax.ShapeDtypeStruct((_NW, _EVENTS), jnp.int32),
    ),
    mesh=plsc.VectorSubcoreMesh(core_axis_name="c", subcore_axis_name="s",
                                num_cores=_NC, num_subcores=_NS),
    scratch_types=(
        pltpu.VMEM((_BLKF, _QTR), jnp.float32),
        pltpu.VMEM((_BLKF, _QTR), jnp.float32),
        pltpu.VMEM((_APW,), jnp.int32),
        pltpu.VMEM((_FPWB + 32,), jnp.int32),
        pltpu.VMEM((_EVENTS,), jnp.int32),
        pltpu.VMEM((_EVENTS,), jnp.int32),
        pltpu.VMEM((_EVENTS, _QTR), jnp.float32),
        pltpu.SemaphoreType.DMA,
        pltpu.SemaphoreType.DMA,
    ),
    compiler_params=pltpu.CompilerParams(needs_layout_passes=False),
)(_sc_body)


def _mm_body(seg_ref, feats_ref, out_ref):
    i = pl.program_id(0)
    seg_blk = seg_ref[0, :]
    onehot = (lax.broadcasted_iota(jnp.int32, (_EVENTS, _BLKT), 0)
              == seg_blk[None, :]).astype(jnp.float32)
    part = lax.dot_general(onehot, feats_ref[...], (((1,), (0,)), ((), ())),
                           preferred_element_type=jnp.float32)

    @pl.when(i == 0)
    def _init():
        out_ref[...] = part

    @pl.when(i > 0)
    def _acc():
        out_ref[...] += part


_tc_segmm = pl.pallas_call(
    _mm_body,
    grid=(_GT,),
    in_specs=[
        pl.BlockSpec((1, _BLKT), lambda i: (0, _OFFB + i)),
        pl.BlockSpec((_BLKT, _HIDDEN), lambda i: (_OFFB + i, 0)),
    ],
    out_specs=pl.BlockSpec((_EVENTS, _HIDDEN), lambda i: (0, 0)),
    out_shape=jax.ShapeDtypeStruct((_EVENTS, _HIDDEN), jnp.float32),
)


def _tc_body(p_ref, mm_ref, sgrid_ref, egrid_ref, q_ref, sim_ref, ts_ref):
    quarters = [None] * _NQ
    for w in range(_NW):
        qi = w % _NQ
        blk = p_ref[w]
        quarters[qi] = blk if quarters[qi] is None else quarters[qi] + blk
    sums = jnp.concatenate(quarters, axis=1) + mm_ref[...]
    inv = 1.0 / jnp.maximum(
        jnp.sqrt(jnp.sum(sums * sums, axis=1, keepdims=True)), 1e-12)
    en = sums * inv
    q = q_ref[...]
    qinv = 1.0 / jnp.maximum(
        jnp.sqrt(jnp.sum(q * q, axis=1, keepdims=True)), 1e-12)
    qn = q * qinv
    sim_ref[...] = lax.dot_general(qn, en, (((1,), (1,)), ((), ())),
                                   preferred_element_type=jnp.float32)
    st = jnp.min(sgrid_ref[...], axis=0, keepdims=True).astype(jnp.float32)
    et = jnp.max(egrid_ref[...], axis=0, keepdims=True).astype(jnp.float32)
    ts_ref[...] = jnp.concatenate([st, et], axis=0).T / 5.0


_tc_finish = pl.pallas_call(
    _tc_body,
    out_shape=[
        jax.ShapeDtypeStruct((64, _EVENTS), jnp.float32),
        jax.ShapeDtypeStruct((_EVENTS, 2), jnp.float32),
    ],
)


@jax.jit
def kernel(query_feats, video_feats, video_events):
    feats = video_feats[0]
    seg = video_events.astype(jnp.int32)
    sums_tc = _tc_segmm(seg.reshape(1, _FRAMES), feats)
    part, sgrid, egrid = _sc_segsum(feats, seg)
    sim, ts = _tc_finish(part, sums_tc, sgrid, egrid, query_feats)
    return sim, ts
